# Initial kernel scaffold; baseline (speedup 1.0000x reference)
import jax
import jax.numpy as jnp
from jax.experimental import pallas as pl


def kernel(v, edge_index, a_values, epsilon, W1, b1, g1, be1, W2, b2, g2, be2):
    raise NotImplementedError("write your pallas kernel here")


# speedup vs baseline: 2.7293x; 2.7293x over previous
import jax
import jax.numpy as jnp
from jax import lax
from jax.experimental import pallas as pl
from jax.experimental.pallas import tpu as pltpu
from jax.experimental.pallas import tpu_sc as plsc

N = 10000
E = 320000
D = 128

NUM_CORES = 2
NUM_SUBCORES = 16
HD = D // NUM_CORES
EDGES_PER_TILE = E // NUM_SUBCORES
CHUNK = 80
CHUNKS_PER_TILE = EDGES_PER_TILE // CHUNK
ZROWS = 624
ZREM = N - NUM_SUBCORES * ZROWS
LANES = 16


def _sc_aggregate_body(col_hbm, row_hbm, a_hbm, v2_hbm, out_hbm,
                       col_v, row_v, a_v, rows_v, zbuf, acc, sem):
    cid = lax.axis_index("c")
    sid = lax.axis_index("s")

    zero16 = jnp.zeros((LANES,), jnp.float32)

    def _zrow(i, _):
        for j in range(HD // LANES):
            zbuf[i, pl.ds(j * LANES, LANES)] = zero16
        return 0

    lax.fori_loop(0, zbuf.shape[0], _zrow, 0)
    for k in range(ZROWS // zbuf.shape[0]):
        pltpu.sync_copy(
            zbuf, acc.at[pl.ds(sid * ZROWS + k * zbuf.shape[0],
                               zbuf.shape[0])])

    @pl.when(sid == NUM_SUBCORES - 1)
    def _zero_tail():
        pltpu.sync_copy(zbuf.at[pl.ds(0, ZREM)],
                        acc.at[pl.ds(NUM_SUBCORES * ZROWS, ZREM)])

    plsc.subcore_barrier()

    pltpu.sync_copy(col_hbm.at[sid], col_v)
    pltpu.sync_copy(row_hbm.at[sid], row_v)
    pltpu.sync_copy(a_hbm.at[pl.ds(sid * EDGES_PER_TILE, EDGES_PER_TILE)], a_v)

    cbase = jnp.full((LANES,), cid * N, jnp.int32)

    def _cadj(i, _):
        for j in range(CHUNK // LANES):
            sl = pl.ds(j * LANES, LANES)
            col_v[i, sl] = col_v[i, sl] + cbase
        return 0

    lax.fori_loop(0, CHUNKS_PER_TILE, _cadj, 0)

    def _chunk(c, _):
        pltpu.async_copy(v2_hbm.at[col_v.at[c]], rows_v, sem).wait()

        def _egroup(eg, _):
            a16 = a_v[pl.ds(c * CHUNK + eg * LANES, LANES)]
            for l in range(LANES):
                ae = a16.at[jnp.full((LANES,), l, jnp.int32)].get(
                    mode="promise_in_bounds")
                e = eg * LANES + l
                for j in range(HD // LANES):
                    sl = pl.ds(j * LANES, LANES)
                    rows_v[e, sl] = rows_v[e, sl] * ae
            return 0

        lax.fori_loop(0, CHUNK // LANES, _egroup, 0)
        pltpu.sync_copy(rows_v, acc.at[row_v.at[c]], add=True)
        return 0

    lax.fori_loop(0, CHUNKS_PER_TILE, _chunk, 0)
    plsc.subcore_barrier()

    pltpu.sync_copy(acc.at[pl.ds(sid * ZROWS, ZROWS)],
                    out_hbm.at[cid, pl.ds(sid * ZROWS, ZROWS)])

    @pl.when(sid == NUM_SUBCORES - 1)
    def _write_tail():
        pltpu.sync_copy(acc.at[pl.ds(NUM_SUBCORES * ZROWS, ZREM)],
                        out_hbm.at[cid, pl.ds(NUM_SUBCORES * ZROWS, ZREM)])


@jax.jit
def _sc_aggregate(col3d, row3d, a_values, v2):
    mesh = plsc.VectorSubcoreMesh(core_axis_name="c", subcore_axis_name="s")
    return pl.kernel(
        _sc_aggregate_body,
        out_type=jax.ShapeDtypeStruct((NUM_CORES, N, HD), jnp.float32),
        mesh=mesh,
        compiler_params=pltpu.CompilerParams(use_tc_tiling_on_sc=False),
        scratch_types=[
            pltpu.VMEM((CHUNKS_PER_TILE, CHUNK), jnp.int32),
            pltpu.VMEM((CHUNKS_PER_TILE, CHUNK), jnp.int32),
            pltpu.VMEM((EDGES_PER_TILE,), jnp.float32),
            pltpu.VMEM((CHUNK, HD), jnp.float32),
            pltpu.VMEM((208, HD), jnp.float32),
            pltpu.VMEM_SHARED((N, HD), jnp.float32),
            pltpu.SemaphoreType.DMA,
        ],
    )(col3d, row3d, a_values, v2)


def _mlp_body(p_ref, v_ref, eps_ref, w1_ref, b1_ref, g1_ref, be1_ref,
              w2_ref, b2_ref, g2_ref, be2_ref, o_ref):
    agg = jnp.concatenate([p_ref[0], p_ref[1]], axis=1)
    agg = agg + eps_ref[0, 0] * v_ref[...]
    h = lax.dot_general(agg, w1_ref[...], (((1,), (1,)), ((), ())),
                        preferred_element_type=jnp.float32,
                        precision=lax.Precision.HIGHEST)
    h = h + b1_ref[...]
    m = jnp.mean(h, axis=0, keepdims=True)
    d = h - m
    var = jnp.mean(d * d, axis=0, keepdims=True)
    h = d * lax.rsqrt(var + 1e-5) * g1_ref[...] + be1_ref[...]
    h = jnp.maximum(h, 0.0)
    o = lax.dot_general(h, w2_ref[...], (((1,), (1,)), ((), ())),
                        preferred_element_type=jnp.float32,
                        precision=lax.Precision.HIGHEST)
    o = o + b2_ref[...]
    m2 = jnp.mean(o, axis=0, keepdims=True)
    d2 = o - m2
    var2 = jnp.mean(d2 * d2, axis=0, keepdims=True)
    o = d2 * lax.rsqrt(var2 + 1e-5) * g2_ref[...] + be2_ref[...]
    o_ref[...] = jnp.maximum(o, 0.0)


@jax.jit
def _mlp(partials, v, epsilon, W1, b1, g1, be1, W2, b2, g2, be2):
    return pl.pallas_call(
        _mlp_body,
        out_shape=jax.ShapeDtypeStruct((N, D), jnp.float32),
    )(partials, v, epsilon,
      W1, b1.reshape(1, -1), g1.reshape(1, -1), be1.reshape(1, -1),
      W2, b2.reshape(1, -1), g2.reshape(1, -1), be2.reshape(1, -1))


def kernel(v, edge_index, a_values, epsilon, W1, b1, g1, be1, W2, b2, g2, be2):
    row3d = edge_index[0].reshape(NUM_SUBCORES, CHUNKS_PER_TILE, CHUNK)
    col3d = edge_index[1].reshape(NUM_SUBCORES, CHUNKS_PER_TILE, CHUNK)
    v2 = jnp.concatenate([v[:, :HD], v[:, HD:]], axis=0)
    partials = _sc_aggregate(col3d, row3d, a_values, v2)
    return _mlp(partials, v, epsilon, W1, b1, g1, be1, W2, b2, g2, be2)


# speedup vs baseline: 3.8325x; 1.4042x over previous
import jax
import jax.numpy as jnp
from jax import lax
from jax.experimental import pallas as pl
from jax.experimental.pallas import tpu as pltpu
from jax.experimental.pallas import tpu_sc as plsc

N = 10000
E = 320000
D = 128

NUM_CORES = 2
NUM_SUBCORES = 16
HD = D // NUM_CORES
EDGES_PER_TILE = E // NUM_SUBCORES
CHUNK = 80
CHUNKS_PER_TILE = EDGES_PER_TILE // CHUNK
ZROWS = 624
ZREM = N - NUM_SUBCORES * ZROWS
LANES = 16


def _sc_aggregate_body(col_hbm, row_hbm, a_hbm, v2_hbm, out_hbm,
                       col_v, row_v, a_v, rows_v, zbuf, acc, sem):
    cid = lax.axis_index("c")
    sid = lax.axis_index("s")

    zero16 = jnp.zeros((LANES,), jnp.float32)

    def _zrow(i, _):
        for j in range(HD // LANES):
            zbuf[i, pl.ds(j * LANES, LANES)] = zero16
        return 0

    lax.fori_loop(0, zbuf.shape[0], _zrow, 0)
    for k in range(ZROWS // zbuf.shape[0]):
        pltpu.sync_copy(
            zbuf, acc.at[pl.ds(sid * ZROWS + k * zbuf.shape[0],
                               zbuf.shape[0])])

    @pl.when(sid == NUM_SUBCORES - 1)
    def _zero_tail():
        pltpu.sync_copy(zbuf.at[pl.ds(0, ZREM)],
                        acc.at[pl.ds(NUM_SUBCORES * ZROWS, ZREM)])

    plsc.subcore_barrier()

    pltpu.sync_copy(col_hbm.at[sid], col_v)
    pltpu.sync_copy(row_hbm.at[sid], row_v)
    pltpu.sync_copy(a_hbm.at[pl.ds(sid * EDGES_PER_TILE, EDGES_PER_TILE)], a_v)

    cbase = jnp.full((LANES,), cid * N, jnp.int32)

    def _cadj(i, _):
        for j in range(CHUNK // LANES):
            sl = pl.ds(j * LANES, LANES)
            col_v[i, sl] = col_v[i, sl] + cbase
        return 0

    lax.fori_loop(0, CHUNKS_PER_TILE, _cadj, 0)

    def _scale(c, buf):
        def _egroup(eg, _):
            a16 = a_v[pl.ds(c * CHUNK + eg * LANES, LANES)]
            for l in range(LANES):
                ae = a16.at[jnp.full((LANES,), l, jnp.int32)].get(
                    mode="promise_in_bounds")
                e = eg * LANES + l
                for j in range(HD // LANES):
                    sl = pl.ds(j * LANES, LANES)
                    buf[e, sl] = buf[e, sl] * ae
            return 0

        lax.fori_loop(0, CHUNK // LANES, _egroup, 0)

    bufs = (rows_v.at[0], rows_v.at[1])
    sems = (sem.at[0], sem.at[1])
    pltpu.async_copy(v2_hbm.at[col_v.at[0]], bufs[0], sems[0])

    def _pair(p, _):
        c0 = 2 * p
        pltpu.async_copy(v2_hbm.at[col_v.at[c0 + 1]], bufs[1], sems[1])
        pltpu.make_async_copy(v2_hbm.at[col_v.at[c0]], bufs[0],
                              sems[0]).wait()
        _scale(c0, bufs[0])
        pltpu.sync_copy(bufs[0], acc.at[row_v.at[c0]], add=True)

        @pl.when(p < CHUNKS_PER_TILE // 2 - 1)
        def _next():
            pltpu.async_copy(v2_hbm.at[col_v.at[c0 + 2]], bufs[0], sems[0])

        pltpu.make_async_copy(v2_hbm.at[col_v.at[c0 + 1]], bufs[1],
                              sems[1]).wait()
        _scale(c0 + 1, bufs[1])
        pltpu.sync_copy(bufs[1], acc.at[row_v.at[c0 + 1]], add=True)
        return 0

    lax.fori_loop(0, CHUNKS_PER_TILE // 2, _pair, 0)
    plsc.subcore_barrier()

    pltpu.sync_copy(acc.at[pl.ds(sid * ZROWS, ZROWS)],
                    out_hbm.at[cid, pl.ds(sid * ZROWS, ZROWS)])

    @pl.when(sid == NUM_SUBCORES - 1)
    def _write_tail():
        pltpu.sync_copy(acc.at[pl.ds(NUM_SUBCORES * ZROWS, ZREM)],
                        out_hbm.at[cid, pl.ds(NUM_SUBCORES * ZROWS, ZREM)])


@jax.jit
def _sc_aggregate(col3d, row3d, a_values, v2):
    mesh = plsc.VectorSubcoreMesh(core_axis_name="c", subcore_axis_name="s")
    return pl.kernel(
        _sc_aggregate_body,
        out_type=jax.ShapeDtypeStruct((NUM_CORES, N, HD), jnp.float32),
        mesh=mesh,
        compiler_params=pltpu.CompilerParams(use_tc_tiling_on_sc=False),
        scratch_types=[
            pltpu.VMEM((CHUNKS_PER_TILE, CHUNK), jnp.int32),
            pltpu.VMEM((CHUNKS_PER_TILE, CHUNK), jnp.int32),
            pltpu.VMEM((EDGES_PER_TILE,), jnp.float32),
            pltpu.VMEM((2, CHUNK, HD), jnp.float32),
            pltpu.VMEM((208, HD), jnp.float32),
            pltpu.VMEM_SHARED((N, HD), jnp.float32),
            pltpu.SemaphoreType.DMA((2,)),
        ],
    )(col3d, row3d, a_values, v2)


def _mlp_body(p_ref, v_ref, eps_ref, w1_ref, b1_ref, g1_ref, be1_ref,
              w2_ref, b2_ref, g2_ref, be2_ref, o_ref):
    agg = jnp.concatenate([p_ref[0], p_ref[1]], axis=1)
    agg = agg + eps_ref[0, 0] * v_ref[...]
    h = lax.dot_general(agg, w1_ref[...], (((1,), (1,)), ((), ())),
                        preferred_element_type=jnp.float32,
                        precision=lax.Precision.HIGHEST)
    h = h + b1_ref[...]
    m = jnp.mean(h, axis=0, keepdims=True)
    d = h - m
    var = jnp.mean(d * d, axis=0, keepdims=True)
    h = d * lax.rsqrt(var + 1e-5) * g1_ref[...] + be1_ref[...]
    h = jnp.maximum(h, 0.0)
    o = lax.dot_general(h, w2_ref[...], (((1,), (1,)), ((), ())),
                        preferred_element_type=jnp.float32,
                        precision=lax.Precision.HIGHEST)
    o = o + b2_ref[...]
    m2 = jnp.mean(o, axis=0, keepdims=True)
    d2 = o - m2
    var2 = jnp.mean(d2 * d2, axis=0, keepdims=True)
    o = d2 * lax.rsqrt(var2 + 1e-5) * g2_ref[...] + be2_ref[...]
    o_ref[...] = jnp.maximum(o, 0.0)


@jax.jit
def _mlp(partials, v, epsilon, W1, b1, g1, be1, W2, b2, g2, be2):
    return pl.pallas_call(
        _mlp_body,
        out_shape=jax.ShapeDtypeStruct((N, D), jnp.float32),
    )(partials, v, epsilon,
      W1, b1.reshape(1, -1), g1.reshape(1, -1), be1.reshape(1, -1),
      W2, b2.reshape(1, -1), g2.reshape(1, -1), be2.reshape(1, -1))


def kernel(v, edge_index, a_values, epsilon, W1, b1, g1, be1, W2, b2, g2, be2):
    row3d = edge_index[0].reshape(NUM_SUBCORES, CHUNKS_PER_TILE, CHUNK)
    col3d = edge_index[1].reshape(NUM_SUBCORES, CHUNKS_PER_TILE, CHUNK)
    v2 = jnp.concatenate([v[:, :HD], v[:, HD:]], axis=0)
    partials = _sc_aggregate(col3d, row3d, a_values, v2)
    return _mlp(partials, v, epsilon, W1, b1, g1, be1, W2, b2, g2, be2)
